# baseline (speedup 1.0000x reference)
import jax
import jax.numpy as jnp
from jax.experimental import pallas as pl
from jax.experimental.pallas import tpu as pltpu

BATCH = 128
R = 64
NUM_ADAPTERS = 64
D_OUT = 4096
BLK = 16
GRID = NUM_ADAPTERS // BLK


def _matmul_body(wids_ref, y_ref, b_ref, out_ref, acc_ref, yd_ref):
    a = pl.program_id(0)

    @pl.when(a == 0)
    def _():
        y2 = y_ref[...] * 2.0
        wids = wids_ref[...]
        for ad in range(NUM_ADAPTERS):
            mask = wids == ad
            y_m = jnp.where(mask, y2, jnp.zeros_like(y2))
            yd_ref[:, ad * R:(ad + 1) * R] = y_m.astype(jnp.bfloat16)

    yd = yd_ref[:, pl.ds(a * (BLK * R), BLK * R)]
    b = b_ref[...].reshape(BLK * R, D_OUT)
    prod = jnp.dot(yd, b, preferred_element_type=jnp.float32)

    @pl.when(a == 0)
    def _():
        acc_ref[...] = prod

    @pl.when((a > 0) & (a < GRID - 1))
    def _():
        acc_ref[...] += prod

    @pl.when(a == GRID - 1)
    def _():
        out_ref[...] = (acc_ref[...] + prod).astype(jnp.bfloat16)


def kernel(y, wids, lora_B):
    y2d = y.reshape(BATCH, R).astype(jnp.float32)
    wids2d = wids.reshape(BATCH, 1)

    out = pl.pallas_call(
        _matmul_body,
        grid=(GRID,),
        in_specs=[
            pl.BlockSpec((BATCH, 1), lambda a: (0, 0)),
            pl.BlockSpec((BATCH, R), lambda a: (0, 0)),
            pl.BlockSpec((BLK, R, D_OUT), lambda a: (a, 0, 0)),
        ],
        out_specs=pl.BlockSpec((BATCH, D_OUT), lambda a: (0, 0)),
        out_shape=jax.ShapeDtypeStruct((BATCH, D_OUT), jnp.bfloat16),
        scratch_shapes=[
            pltpu.VMEM((BATCH, D_OUT), jnp.float32),
            pltpu.VMEM((BATCH, NUM_ADAPTERS * R), jnp.bfloat16),
        ],
        compiler_params=pltpu.CompilerParams(
            dimension_semantics=("arbitrary",),
        ),
    )(wids2d, y2d, lora_B.astype(jnp.bfloat16))
    return out.astype(y.dtype).reshape(BATCH, 1, D_OUT)


# speedup vs baseline: 1.0638x; 1.0638x over previous
import jax
import jax.numpy as jnp
from jax.experimental import pallas as pl
from jax.experimental.pallas import tpu as pltpu

BATCH = 128
R = 64
NUM_ADAPTERS = 64
D_OUT = 4096
BLK = 8
GRID = NUM_ADAPTERS // BLK
NBUF = 4


def _matmul_body(wids_ref, y_ref, b_hbm, out_ref, acc_ref, yd_ref,
                 bufs, sems):
    a = pl.program_id(0)

    def _copy(step):
        return pltpu.make_async_copy(
            b_hbm.at[pl.ds(step * BLK, BLK)],
            bufs.at[step % NBUF],
            sems.at[step % NBUF])

    @pl.when(a == 0)
    def _():
        for i in range(NBUF - 1):
            _copy(i).start()
        y2 = y_ref[...] * 2.0
        wids = wids_ref[...]
        for ad in range(NUM_ADAPTERS):
            mask = wids == ad
            y_m = jnp.where(mask, y2, jnp.zeros_like(y2))
            yd_ref[:, ad * R:(ad + 1) * R] = y_m.astype(jnp.bfloat16)

    @pl.when(a + NBUF - 1 < GRID)
    def _():
        _copy(a + NBUF - 1).start()

    _copy(a).wait()

    yd = yd_ref[:, pl.ds(a * (BLK * R), BLK * R)]
    b = bufs[a % NBUF].reshape(BLK * R, D_OUT)
    prod = jnp.dot(yd, b, preferred_element_type=jnp.float32)

    @pl.when(a == 0)
    def _():
        acc_ref[...] = prod

    @pl.when((a > 0) & (a < GRID - 1))
    def _():
        acc_ref[...] += prod

    @pl.when(a == GRID - 1)
    def _():
        out_ref[...] = (acc_ref[...] + prod).astype(jnp.bfloat16)


def kernel(y, wids, lora_B):
    y2d = y.reshape(BATCH, R).astype(jnp.float32)
    wids2d = wids.reshape(BATCH, 1)

    out = pl.pallas_call(
        _matmul_body,
        grid=(GRID,),
        in_specs=[
            pl.BlockSpec((BATCH, 1), lambda a: (0, 0)),
            pl.BlockSpec((BATCH, R), lambda a: (0, 0)),
            pl.BlockSpec(memory_space=pl.ANY),
        ],
        out_specs=pl.BlockSpec((BATCH, D_OUT), lambda a: (0, 0)),
        out_shape=jax.ShapeDtypeStruct((BATCH, D_OUT), jnp.bfloat16),
        scratch_shapes=[
            pltpu.VMEM((BATCH, D_OUT), jnp.float32),
            pltpu.VMEM((BATCH, NUM_ADAPTERS * R), jnp.bfloat16),
            pltpu.VMEM((NBUF, BLK, R, D_OUT), jnp.bfloat16),
            pltpu.SemaphoreType.DMA((NBUF,)),
        ],
        compiler_params=pltpu.CompilerParams(
            dimension_semantics=("arbitrary",),
        ),
    )(wids2d, y2d, lora_B.astype(jnp.bfloat16))
    return out.astype(y.dtype).reshape(BATCH, 1, D_OUT)
